# Initial kernel scaffold; baseline (speedup 1.0000x reference)
import jax
import jax.numpy as jnp
from jax.experimental import pallas as pl


def kernel(x_t, x_t1, edge_index, edge_attr):
    raise NotImplementedError("write your pallas kernel here")


# speedup vs baseline: 88.8255x; 88.8255x over previous
import functools

import jax
import jax.numpy as jnp
from jax import lax
from jax.experimental import pallas as pl
from jax.experimental.pallas import tpu as pltpu
from jax.experimental.pallas import tpu_sc as plsc

N = 10000
E = 320000
NT = 16
NPAD = 10240
EPT = E // NT
CHUNK = NPAD // NT
L = 16
DELTA_T = 0.01
MU = 0.01

_mesh = plsc.VectorSubcoreMesh(
    core_axis_name="c", subcore_axis_name="s", num_cores=1
)


@functools.partial(
    pl.kernel,
    mesh=_mesh,
    out_type=jax.ShapeDtypeStruct((NPAD,), jnp.float32),
    scratch_types=[
        pltpu.VMEM((NPAD,), jnp.float32),
        pltpu.VMEM((EPT,), jnp.int32),
        pltpu.VMEM((EPT,), jnp.int32),
        pltpu.VMEM((EPT,), jnp.float32),
        pltpu.VMEM((NPAD,), jnp.float32),
        pltpu.VMEM((NPAD,), jnp.float32),
        pltpu.VMEM((CHUNK,), jnp.float32),
        pltpu.VMEM((CHUNK,), jnp.float32),
        pltpu.VMEM((CHUNK,), jnp.float32),
        pltpu.VMEM((CHUNK,), jnp.float32),
        pltpu.VMEM((CHUNK,), jnp.float32),
        pltpu.VMEM((CHUNK,), jnp.float32),
        pltpu.VMEM((CHUNK,), jnp.float32),
        pltpu.VMEM_SHARED((NT, NPAD), jnp.float32),
        pltpu.VMEM_SHARED((NT, NPAD), jnp.float32),
        pltpu.VMEM_SHARED((NPAD,), jnp.float32),
    ],
    compiler_params=pltpu.CompilerParams(needs_layout_passes=False),
)
def _sc_loss(u_t_hbm, u_t1_hbm, src_hbm, dst_hbm, ea_hbm, out_hbm,
             u1_full, src_v, dst_v, ea_v, acc_s, acc_c,
             tmp_c, sum_c, cnt_c, spat_c, ut_c, u1_c, loss_c,
             sum_slots, cnt_slots, spat_shared):
    tid = lax.axis_index("s")
    ebase = pl.multiple_of(tid * EPT, 8)
    nbase = pl.multiple_of(tid * CHUNK, 8)

    pltpu.sync_copy(u_t1_hbm, u1_full)
    pltpu.sync_copy(src_hbm.at[pl.ds(ebase, EPT)], src_v)
    pltpu.sync_copy(dst_hbm.at[pl.ds(ebase, EPT)], dst_v)
    pltpu.sync_copy(ea_hbm.at[pl.ds(ebase, EPT)], ea_v)
    pltpu.sync_copy(u_t_hbm.at[pl.ds(nbase, CHUNK)], ut_c)
    pltpu.sync_copy(u_t1_hbm.at[pl.ds(nbase, CHUNK)], u1_c)

    zeros = jnp.zeros((L,), jnp.float32)
    ones = jnp.ones((L,), jnp.float32)

    def zero_body(i, _):
        sl = pl.ds(pl.multiple_of(i * L, L), L)
        acc_s[sl] = zeros
        acc_c[sl] = zeros
        return 0
    lax.fori_loop(0, NPAD // L, zero_body, 0)

    def edge1(i, _):
        sl = pl.ds(pl.multiple_of(i * L, L), L)
        s = src_v[sl]
        d = dst_v[sl]
        ea = ea_v[sl]
        us = plsc.load_gather(u1_full, [s])
        ud = plsc.load_gather(u1_full, [d])
        val = (ud - us) / ea
        plsc.addupdate_scatter(acc_s, [d], val)
        plsc.addupdate_scatter(acc_c, [d], ones)
        return 0
    lax.fori_loop(0, EPT // L, edge1, 0)

    pltpu.sync_copy(acc_s, sum_slots.at[tid])
    pltpu.sync_copy(acc_c, cnt_slots.at[tid])
    plsc.subcore_barrier()

    def zero_chunk(i, _):
        sl = pl.ds(pl.multiple_of(i * L, L), L)
        sum_c[sl] = zeros
        cnt_c[sl] = zeros
        return 0
    lax.fori_loop(0, CHUNK // L, zero_chunk, 0)

    def red1(p, _):
        pltpu.sync_copy(sum_slots.at[p, pl.ds(nbase, CHUNK)], tmp_c)

        def add_body(j, _):
            sl = pl.ds(pl.multiple_of(j * L, L), L)
            sum_c[sl] = sum_c[sl] + tmp_c[sl]
            return 0
        lax.fori_loop(0, CHUNK // L, add_body, 0)

        pltpu.sync_copy(cnt_slots.at[p, pl.ds(nbase, CHUNK)], tmp_c)

        def add_body2(j, _):
            sl = pl.ds(pl.multiple_of(j * L, L), L)
            cnt_c[sl] = cnt_c[sl] + tmp_c[sl]
            return 0
        lax.fori_loop(0, CHUNK // L, add_body2, 0)
        return 0
    lax.fori_loop(0, NT, red1, 0)

    def finish_spatial(j, _):
        sl = pl.ds(pl.multiple_of(j * L, L), L)
        spat_c[sl] = sum_c[sl] / jnp.maximum(cnt_c[sl], 1.0)
        return 0
    lax.fori_loop(0, CHUNK // L, finish_spatial, 0)

    pltpu.sync_copy(spat_c, spat_shared.at[pl.ds(nbase, CHUNK)])
    plsc.subcore_barrier()
    pltpu.sync_copy(spat_shared, u1_full)

    def zero2(i, _):
        sl = pl.ds(pl.multiple_of(i * L, L), L)
        acc_s[sl] = zeros
        return 0
    lax.fori_loop(0, NPAD // L, zero2, 0)

    def edge2(i, _):
        sl = pl.ds(pl.multiple_of(i * L, L), L)
        s = src_v[sl]
        d = dst_v[sl]
        ea = ea_v[sl]
        ss = plsc.load_gather(u1_full, [s])
        sd = plsc.load_gather(u1_full, [d])
        val = (sd - ss) / ea
        plsc.addupdate_scatter(acc_s, [d], val)
        return 0
    lax.fori_loop(0, EPT // L, edge2, 0)

    pltpu.sync_copy(acc_s, sum_slots.at[tid])
    plsc.subcore_barrier()

    def zero_chunk2(i, _):
        sl = pl.ds(pl.multiple_of(i * L, L), L)
        sum_c[sl] = zeros
        return 0
    lax.fori_loop(0, CHUNK // L, zero_chunk2, 0)

    def red2(p, _):
        pltpu.sync_copy(sum_slots.at[p, pl.ds(nbase, CHUNK)], tmp_c)

        def add_body(j, _):
            sl = pl.ds(pl.multiple_of(j * L, L), L)
            sum_c[sl] = sum_c[sl] + tmp_c[sl]
            return 0
        lax.fori_loop(0, CHUNK // L, add_body, 0)
        return 0
    lax.fori_loop(0, NT, red2, 0)

    def finish(j, _):
        sl = pl.ds(pl.multiple_of(j * L, L), L)
        second = sum_c[sl] / jnp.maximum(cnt_c[sl], 1.0)
        temporal = (ut_c[sl] - u1_c[sl]) * (1.0 / DELTA_T)
        loss_c[sl] = temporal + spat_c[sl] * u1_c[sl] - MU * second
        return 0
    lax.fori_loop(0, CHUNK // L, finish, 0)

    pltpu.sync_copy(loss_c, out_hbm.at[pl.ds(nbase, CHUNK)])


def kernel(x_t, x_t1, edge_index, edge_attr):
    u_t = jnp.pad(x_t[:, 0], (0, NPAD - N))
    u_t1 = jnp.pad(x_t1[:, 0], (0, NPAD - N))
    src = edge_index[0]
    dst = edge_index[1]
    ea = edge_attr[:, 0]
    out = _sc_loss(u_t, u_t1, src, dst, ea)
    return out[:N]


# speedup vs baseline: 133.1914x; 1.4995x over previous
import functools

import jax
import jax.numpy as jnp
from jax import lax
from jax.experimental import pallas as pl
from jax.experimental.pallas import tpu as pltpu
from jax.experimental.pallas import tpu_sc as plsc

N = 10000
E = 320000
NT = 16
NPAD = 10240
EPT = E // NT
CHUNK = NPAD // NT
L = 16
DELTA_T = 0.01
MU = 0.01

_mesh = plsc.VectorSubcoreMesh(
    core_axis_name="c", subcore_axis_name="s", num_cores=1
)


@functools.partial(
    pl.kernel,
    mesh=_mesh,
    out_type=jax.ShapeDtypeStruct((NPAD,), jnp.float32),
    scratch_types=[
        pltpu.VMEM((NPAD,), jnp.float32),
        pltpu.VMEM((EPT,), jnp.int32),
        pltpu.VMEM((EPT,), jnp.int32),
        pltpu.VMEM((EPT,), jnp.float32),
        pltpu.VMEM((NPAD,), jnp.float32),
        pltpu.VMEM((NPAD,), jnp.float32),
        pltpu.VMEM((NT, CHUNK), jnp.float32),
        pltpu.VMEM((CHUNK,), jnp.float32),
        pltpu.VMEM((CHUNK,), jnp.float32),
        pltpu.VMEM((CHUNK,), jnp.float32),
        pltpu.VMEM((CHUNK,), jnp.float32),
        pltpu.VMEM((CHUNK,), jnp.float32),
        pltpu.SemaphoreType.DMA,
        pltpu.VMEM_SHARED((NT, NPAD), jnp.float32),
        pltpu.VMEM_SHARED((NT, NPAD), jnp.float32),
        pltpu.VMEM_SHARED((NPAD,), jnp.float32),
    ],
    compiler_params=pltpu.CompilerParams(needs_layout_passes=False),
)
def _sc_loss(u_t_hbm, u_t1_hbm, src_hbm, dst_hbm, ea_hbm, out_hbm,
             u1_full, src_v, dst_v, ea_v, acc_s, acc_c,
             part_s, cnt_c, spat_c, ut_c, u1_c, loss_c,
             dma_sem, sum_slots, cnt_slots, spat_shared):
    tid = lax.axis_index("s")
    ebase = pl.multiple_of(tid * EPT, 8)
    nbase = pl.multiple_of(tid * CHUNK, 8)

    zeros = jnp.zeros((L,), jnp.float32)
    ones = jnp.ones((L,), jnp.float32)

    wave1 = [
        pltpu.async_copy(u_t1_hbm, u1_full, dma_sem),
        pltpu.async_copy(src_hbm.at[pl.ds(ebase, EPT)], src_v, dma_sem),
        pltpu.async_copy(ea_hbm.at[pl.ds(ebase, EPT)], ea_v, dma_sem),
    ]

    @plsc.parallel_loop(0, NPAD // L, unroll=4)
    def _zero1(i):
        sl = pl.ds(pl.multiple_of(i * L, L), L)
        acc_s[sl] = zeros
        acc_c[sl] = zeros

    for c in wave1:
        c.wait()

    wave2 = [
        pltpu.async_copy(dst_hbm.at[pl.ds(ebase, EPT)], dst_v, dma_sem),
        pltpu.async_copy(u_t_hbm.at[pl.ds(nbase, CHUNK)], ut_c, dma_sem),
        pltpu.async_copy(u_t1_hbm.at[pl.ds(nbase, CHUNK)], u1_c, dma_sem),
    ]

    @plsc.parallel_loop(0, EPT // L, unroll=4)
    def _recip0(i):
        sl = pl.ds(pl.multiple_of(i * L, L), L)
        ea_v[sl] = 1.0 / ea_v[sl]

    for c in wave2:
        c.wait()

    @plsc.parallel_loop(0, EPT // L, unroll=4)
    def _edge1(i):
        sl = pl.ds(pl.multiple_of(i * L, L), L)
        s = src_v[sl]
        d = dst_v[sl]
        r = ea_v[sl]
        us = plsc.load_gather(u1_full, [s])
        ud = plsc.load_gather(u1_full, [d])
        val = (ud - us) * r
        plsc.addupdate_scatter(acc_s, [d], val)
        plsc.addupdate_scatter(acc_c, [d], ones)

    pltpu.sync_copy(acc_s, sum_slots.at[tid])
    pltpu.sync_copy(acc_c, cnt_slots.at[tid])

    @plsc.parallel_loop(0, NPAD // L, unroll=4)
    def _zero2(i):
        sl = pl.ds(pl.multiple_of(i * L, L), L)
        acc_s[sl] = zeros

    plsc.subcore_barrier()

    pltpu.sync_copy(sum_slots.at[:, pl.ds(nbase, CHUNK)], part_s)

    @plsc.parallel_loop(0, CHUNK // L, unroll=2)
    def _red1s(j):
        sl = pl.ds(pl.multiple_of(j * L, L), L)
        s = part_s[0, sl]
        for p in range(1, NT):
            s = s + part_s[p, sl]
        spat_c[sl] = s

    pltpu.sync_copy(cnt_slots.at[:, pl.ds(nbase, CHUNK)], part_s)

    @plsc.parallel_loop(0, CHUNK // L, unroll=2)
    def _red1c(j):
        sl = pl.ds(pl.multiple_of(j * L, L), L)
        c = part_s[0, sl]
        for p in range(1, NT):
            c = c + part_s[p, sl]
        c = jnp.maximum(c, 1.0)
        cnt_c[sl] = c
        spat_c[sl] = spat_c[sl] / c

    pltpu.sync_copy(spat_c, spat_shared.at[pl.ds(nbase, CHUNK)])
    plsc.subcore_barrier()
    pltpu.sync_copy(spat_shared, u1_full)

    @plsc.parallel_loop(0, EPT // L, unroll=4)
    def _edge2(i):
        sl = pl.ds(pl.multiple_of(i * L, L), L)
        s = src_v[sl]
        d = dst_v[sl]
        r = ea_v[sl]
        ss = plsc.load_gather(u1_full, [s])
        sd = plsc.load_gather(u1_full, [d])
        val = (sd - ss) * r
        plsc.addupdate_scatter(acc_s, [d], val)

    pltpu.sync_copy(acc_s, sum_slots.at[tid])
    plsc.subcore_barrier()

    pltpu.sync_copy(sum_slots.at[:, pl.ds(nbase, CHUNK)], part_s)

    @plsc.parallel_loop(0, CHUNK // L, unroll=2)
    def _finish(j):
        sl = pl.ds(pl.multiple_of(j * L, L), L)
        s2 = part_s[0, sl]
        for p in range(1, NT):
            s2 = s2 + part_s[p, sl]
        second = s2 / cnt_c[sl]
        temporal = (ut_c[sl] - u1_c[sl]) * (1.0 / DELTA_T)
        loss_c[sl] = temporal + spat_c[sl] * u1_c[sl] - MU * second

    pltpu.sync_copy(loss_c, out_hbm.at[pl.ds(nbase, CHUNK)])


def kernel(x_t, x_t1, edge_index, edge_attr):
    u_t = jnp.pad(x_t[:, 0], (0, NPAD - N))
    u_t1 = jnp.pad(x_t1[:, 0], (0, NPAD - N))
    src = edge_index[0]
    dst = edge_index[1]
    ea = edge_attr[:, 0]
    out = _sc_loss(u_t, u_t1, src, dst, ea)
    return out[:N]
